# Initial kernel scaffold; baseline (speedup 1.0000x reference)
import jax
import jax.numpy as jnp
from jax.experimental import pallas as pl


def kernel(imgs, annotations, regression, classification, anchors):
    raise NotImplementedError("write your pallas kernel here")


# speedup vs baseline: 22.1506x; 22.1506x over previous
import functools

import jax
import jax.numpy as jnp
from jax import lax
from jax.experimental import pallas as pl
from jax.experimental.pallas import tpu as pltpu

B, N, C = 4, 20000, 80
IMG = 512.0
PRE_NMS_THRESH = 0.05
NMS_THRESH = 0.5
TOP_N = 100

NP = 20480
BL = 2048
SR = 160
SC = 128
NEG = float("-inf")


def _stage1(cls_ref, reg_ref, anc_ref,
            s_ref, ox1_ref, oy1_ref, ox2_ref, oy2_ref,
            x1_ref, y1_ref, x2_ref, y2_ref, cf_ref):
    x = cls_ref[0]
    m = jnp.max(x, axis=0)
    am = jnp.argmax(x, axis=0)
    s = jnp.where(m > PRE_NMS_THRESH, m, NEG)

    r = reg_ref[0]
    a = anc_ref[0]
    a0, a1, a2, a3 = a[0], a[1], a[2], a[3]
    r0, r1, r2, r3 = r[0], r[1], r[2], r[3]
    y_c_a = (a0 + a2) / 2.0
    x_c_a = (a1 + a3) / 2.0
    ha = a2 - a0
    wa = a3 - a1
    w = jnp.exp(r3) * wa
    h = jnp.exp(r2) * ha
    y_c = r0 * ha + y_c_a
    x_c = r1 * wa + x_c_a
    x1 = jnp.clip(x_c - w / 2.0, 0.0, IMG)
    y1 = jnp.clip(y_c - h / 2.0, 0.0, IMG)
    x2 = jnp.clip(x_c + w / 2.0, 0.0, IMG)
    y2 = jnp.clip(y_c + h / 2.0, 0.0, IMG)

    off = am.astype(jnp.float32) * (2.0 * IMG)
    s_ref[0, 0] = s
    ox1_ref[0, 0] = x1 + off
    oy1_ref[0, 0] = y1 + off
    ox2_ref[0, 0] = x2 + off
    oy2_ref[0, 0] = y2 + off
    x1_ref[0, 0] = x1
    y1_ref[0, 0] = y1
    x2_ref[0, 0] = x2
    y2_ref[0, 0] = y2
    cf_ref[0, 0] = am.astype(jnp.float32)


def _stage2(s_in, ox1_ref, oy1_ref, ox2_ref, oy2_ref,
            x1_ref, y1_ref, x2_ref, y2_ref, cf_ref,
            packed_ref, s_ref):
    s_ref[...] = s_in[...]
    fio = (lax.broadcasted_iota(jnp.int32, (SR, SC), 0) * SC
           + lax.broadcasted_iota(jnp.int32, (SR, SC), 1))
    lane = lax.broadcasted_iota(jnp.int32, (1, SC), 1)

    def maxidx(sb):
        m = jnp.max(sb)
        idx = jnp.min(jnp.where(sb == m, fio, jnp.int32(1 << 30)))
        return m, idx

    init = []
    for b in range(B):
        m, i = maxidx(s_ref[pl.ds(b * SR, SR), :])
        init += [m, i]

    def body(it, carry):
        nxt = []
        vec = jnp.zeros((1, SC), jnp.float32)
        for b in range(B):
            m, i = carry[2 * b], carry[2 * b + 1]
            valid = m > NEG
            sl = pl.ds(b * SR, SR)
            eq = fio == i

            def pick(ref):
                return jnp.max(jnp.where(eq, ref[sl, :], NEG))

            wox1 = pick(ox1_ref)
            woy1 = pick(oy1_ref)
            wox2 = pick(ox2_ref)
            woy2 = pick(oy2_ref)
            outs = [jnp.where(valid, pick(x1_ref), 0.0),
                    jnp.where(valid, pick(y1_ref), 0.0),
                    jnp.where(valid, pick(x2_ref), 0.0),
                    jnp.where(valid, pick(y2_ref), 0.0),
                    jnp.where(valid, m, 0.0),
                    jnp.where(valid, pick(cf_ref), -1.0)]
            for q, v in enumerate(outs):
                vec = jnp.where(lane == b * 8 + q, v, vec)

            bx1 = ox1_ref[sl, :]
            by1 = oy1_ref[sl, :]
            bx2 = ox2_ref[sl, :]
            by2 = oy2_ref[sl, :]
            xx1 = jnp.maximum(wox1, bx1)
            yy1 = jnp.maximum(woy1, by1)
            xx2 = jnp.minimum(wox2, bx2)
            yy2 = jnp.minimum(woy2, by2)
            inter = jnp.maximum(xx2 - xx1, 0.0) * jnp.maximum(yy2 - yy1, 0.0)
            a1 = (jnp.maximum(wox2 - wox1, 0.0)
                  * jnp.maximum(woy2 - woy1, 0.0))
            a2 = (jnp.maximum(bx2 - bx1, 0.0)
                  * jnp.maximum(by2 - by1, 0.0))
            iou = inter / (a1 + a2 - inter + 1e-8)
            sb = s_ref[sl, :]
            sb = jnp.where(iou > NMS_THRESH, NEG, sb)
            s_ref[sl, :] = sb
            m2, i2 = maxidx(sb)
            nxt += [m2, i2]
        packed_ref[pl.ds(it, 1), :] = vec
        return tuple(nxt)

    lax.fori_loop(0, TOP_N, body, tuple(init))


@jax.jit
def kernel(imgs, annotations, regression, classification, anchors):
    del imgs, annotations
    cls_p = jnp.pad(classification, ((0, 0), (0, NP - N), (0, 0)),
                    constant_values=-1.0).transpose(0, 2, 1)
    reg_p = jnp.pad(regression, ((0, 0), (0, NP - N), (0, 0))
                    ).transpose(0, 2, 1)
    anc_p = jnp.pad(anchors, ((0, 0), (0, NP - N), (0, 0))
                    ).transpose(0, 2, 1)

    plane = jax.ShapeDtypeStruct((B, 1, NP), jnp.float32)
    planes = pl.pallas_call(
        _stage1,
        grid=(B, NP // BL),
        in_specs=[
            pl.BlockSpec((1, C, BL), lambda b, n: (b, 0, n)),
            pl.BlockSpec((1, 4, BL), lambda b, n: (b, 0, n)),
            pl.BlockSpec((1, 4, BL), lambda b, n: (0, 0, n)),
        ],
        out_specs=[pl.BlockSpec((1, 1, BL), lambda b, n: (b, 0, n))] * 10,
        out_shape=[plane] * 10,
    )(cls_p, reg_p, anc_p)

    slabs = [p.reshape(B * SR, SC) for p in planes]

    packed = pl.pallas_call(
        _stage2,
        out_shape=jax.ShapeDtypeStruct((TOP_N, SC), jnp.float32),
        scratch_shapes=[pltpu.VMEM((B * SR, SC), jnp.float32)],
    )(*slabs)
    boxes = jnp.stack([packed[:, b * 8:b * 8 + 4] for b in range(B)])
    scores = jnp.stack([packed[:, b * 8 + 4] for b in range(B)])
    classes = jnp.stack([packed[:, b * 8 + 5] for b in range(B)]).astype(jnp.int32)
    return boxes, scores, classes
